# Initial kernel scaffold; baseline (speedup 1.0000x reference)
import functools

import jax
import jax.numpy as jnp
from jax import lax
from jax.experimental import pallas as pl
from jax.experimental.pallas import tpu as pltpu
from jax.experimental.pallas import tpu_sc as plsc

N = 10000
E = 320000
D = 128
F = 128

NC = 2
NS = 16
WIN = 80
NWIN = E // WIN
WIN_PER_TILE_A = NWIN // NS
WIN_PER_TILE_C = NWIN // (NC * NS)
NPAD = 10240
ROWS_PER_TILE = N // NS

_mesh = plsc.VectorSubcoreMesh(core_axis_name="c", subcore_axis_name="s")


@functools.partial(
    pl.kernel,
    out_type=jax.ShapeDtypeStruct((2, N), jnp.float32),
    mesh=_mesh,
    scratch_types=[
        pltpu.VMEM_SHARED((NPAD,), jnp.float32),
        pltpu.VMEM((WIN_PER_TILE_A, WIN), jnp.int32),
        pltpu.VMEM((WIN,), jnp.float32),
        pltpu.VMEM((640,), jnp.float32),
        pltpu.SemaphoreType.DMA((2,)),
    ],
)
def _degree_kernel(e3, deg, hist, idxbuf, ones, zbuf, sems):
    c = lax.axis_index("c")
    s = lax.axis_index("s")

    @pl.loop(0, 640 // 16)
    def _fill_z(i):
        zbuf[pl.ds(i * 16, 16)] = jnp.zeros((16,), jnp.float32)

    @pl.loop(0, WIN // 16)
    def _fill_o(i):
        ones[pl.ds(i * 16, 16)] = jnp.ones((16,), jnp.float32)

    pltpu.sync_copy(zbuf, hist.at[pl.ds(s * 640, 640)])
    pltpu.sync_copy(e3.at[c, pl.ds(s * WIN_PER_TILE_A, WIN_PER_TILE_A)], idxbuf)
    plsc.subcore_barrier()

    @pl.loop(0, WIN_PER_TILE_A, step=2)
    def _scat(j):
        cp0 = pltpu.async_copy(ones, hist.at[idxbuf.at[j]], sems.at[0], add=True)
        cp1 = pltpu.async_copy(ones, hist.at[idxbuf.at[j + 1]], sems.at[1], add=True)
        cp0.wait()
        cp1.wait()

    plsc.subcore_barrier()

    @pl.when(s < NS - 1)
    def _wb():
        pltpu.sync_copy(hist.at[pl.ds(s * 640, 640)], deg.at[c, pl.ds(s * 640, 640)])

    @pl.when(s == NS - 1)
    def _wb_last():
        pltpu.sync_copy(hist.at[pl.ds(9600, 400)], deg.at[c, pl.ds(9600, 400)])


@functools.partial(
    pl.kernel,
    out_type=jax.ShapeDtypeStruct((NC, N, F), jnp.float32),
    mesh=_mesh,
    scratch_types=[
        pltpu.VMEM_SHARED((N, F), jnp.float32),
        pltpu.VMEM((WIN_PER_TILE_C, WIN), jnp.int32),
        pltpu.VMEM((WIN_PER_TILE_C, WIN), jnp.int32),
        pltpu.VMEM((2, WIN, F), jnp.float32),
        pltpu.SemaphoreType.DMA((2,)),
        pltpu.SemaphoreType.DMA((2,)),
    ],
)
def _aggregate_kernel(h2, e3, zeros_h, acc_out, accs, sidx, didx, rbuf, gsem, ssem):
    c = lax.axis_index("c")
    s = lax.axis_index("s")
    base = c * (NWIN // NC) + s * WIN_PER_TILE_C

    pltpu.sync_copy(zeros_h.at[pl.ds(s * ROWS_PER_TILE, ROWS_PER_TILE)],
                    accs.at[pl.ds(s * ROWS_PER_TILE, ROWS_PER_TILE)])
    pltpu.sync_copy(e3.at[0, pl.ds(base, WIN_PER_TILE_C)], sidx)
    pltpu.sync_copy(e3.at[1, pl.ds(base, WIN_PER_TILE_C)], didx)
    plsc.subcore_barrier()

    pltpu.async_copy(h2.at[sidx.at[0]], rbuf.at[0], gsem.at[0])

    @pl.loop(0, WIN_PER_TILE_C, step=2)
    def _edges(j):
        for b in range(2):
            jj = j + b
            nb = 1 - b

            @pl.when(jj + 1 < WIN_PER_TILE_C)
            def _prefetch():
                @pl.when(jj >= 1)
                def _drain_prev():
                    pltpu.make_async_copy(
                        rbuf.at[nb], accs.at[didx.at[jj - 1]], ssem.at[nb]
                    ).wait()

                pltpu.async_copy(h2.at[sidx.at[jj + 1]], rbuf.at[nb], gsem.at[nb])

            pltpu.make_async_copy(h2.at[sidx.at[jj]], rbuf.at[b], gsem.at[b]).wait()
            pltpu.async_copy(rbuf.at[b], accs.at[didx.at[jj]], ssem.at[b], add=True)

    last = WIN_PER_TILE_C - 1
    pltpu.make_async_copy(rbuf.at[0], accs.at[didx.at[last - 1]], ssem.at[0]).wait()
    pltpu.make_async_copy(rbuf.at[1], accs.at[didx.at[last]], ssem.at[1]).wait()

    plsc.subcore_barrier()
    pltpu.sync_copy(accs.at[pl.ds(s * ROWS_PER_TILE, ROWS_PER_TILE)],
                    acc_out.at[c, pl.ds(s * ROWS_PER_TILE, ROWS_PER_TILE)])


def _matmul_body(x_ref, w_ref, d_ref, o_ref):
    m = jnp.dot(x_ref[...], w_ref[...], preferred_element_type=jnp.float32)
    scale = lax.rsqrt(jnp.maximum(d_ref[...], 1.0))
    o_ref[...] = m * scale


def _scaled_matmul(x, w, deg_out):
    blk = 500
    grid = N // blk
    return pl.pallas_call(
        _matmul_body,
        grid=(grid,),
        in_specs=[
            pl.BlockSpec((blk, D), lambda i: (i, 0)),
            pl.BlockSpec((D, F), lambda i: (0, 0)),
            pl.BlockSpec((blk, 1), lambda i: (i, 0)),
        ],
        out_specs=pl.BlockSpec((blk, F), lambda i: (i, 0)),
        out_shape=jax.ShapeDtypeStruct((N, F), jnp.float32),
    )(x, w, deg_out)


def _final_body(a_ref, d_ref, o_ref):
    a = a_ref[0] + a_ref[1]
    scale = lax.rsqrt(jnp.maximum(d_ref[...], 1.0))
    o_ref[...] = jnp.maximum(a * scale, 0.0)


def _finalize(acc, deg_in):
    blk = 500
    grid = N // blk
    return pl.pallas_call(
        _final_body,
        grid=(grid,),
        in_specs=[
            pl.BlockSpec((2, blk, F), lambda i: (0, i, 0)),
            pl.BlockSpec((blk, 1), lambda i: (i, 0)),
        ],
        out_specs=pl.BlockSpec((blk, F), lambda i: (i, 0)),
        out_shape=jax.ShapeDtypeStruct((N, F), jnp.float32),
    )(acc, deg_in)


def kernel(x, edge_index, W):
    e3 = edge_index.reshape(2, NWIN, WIN)
    deg = _degree_kernel(e3)
    h2 = _scaled_matmul(x, W, deg[0].reshape(N, 1))
    zeros_h = jnp.zeros((N, F), jnp.float32)
    acc = _aggregate_kernel(h2, e3, zeros_h)
    return _finalize(acc, deg[1].reshape(N, 1))


# speedup vs baseline: 24.0776x; 24.0776x over previous
import functools

import jax
import jax.numpy as jnp
from jax import lax
from jax.experimental import pallas as pl
from jax.experimental.pallas import tpu as pltpu
from jax.experimental.pallas import tpu_sc as plsc

N = 10000
E = 320000
D = 128
F = 128

NC = 2
NS = 16
WIN = 80
NWIN = E // WIN
WIN_PER_TILE_A = NWIN // NS
WIN_PER_TILE_C = NWIN // (NC * NS)
NPAD = 10240
ROWS_PER_TILE = N // NS

_mesh = plsc.VectorSubcoreMesh(core_axis_name="c", subcore_axis_name="s",
                               num_cores=NC, num_subcores=NS)


_DEG_KERNEL_ARGS = dict(
    out_type=jax.ShapeDtypeStruct((2, N), jnp.float32),
    mesh=_mesh,
    scratch_types=[
        pltpu.VMEM_SHARED((NPAD,), jnp.float32),
        pltpu.VMEM((WIN_PER_TILE_A, WIN), jnp.int32),
        pltpu.VMEM((WIN,), jnp.float32),
        pltpu.VMEM((640,), jnp.float32),
        pltpu.SemaphoreType.DMA((2,)),
    ],
    compiler_params=pltpu.CompilerParams(use_tc_tiling_on_sc=False),
)


def _degree_body(e3, deg, hist, idxbuf, ones, zbuf, sems):
    c = lax.axis_index("c")
    s = lax.axis_index("s")

    @pl.loop(0, 640 // 16)
    def _fill_z(i):
        zbuf[pl.ds(i * 16, 16)] = jnp.zeros((16,), jnp.float32)

    @pl.loop(0, WIN // 16)
    def _fill_o(i):
        ones[pl.ds(i * 16, 16)] = jnp.ones((16,), jnp.float32)

    pltpu.sync_copy(zbuf, hist.at[pl.ds(s * 640, 640)])
    pltpu.sync_copy(e3.at[c, pl.ds(s * WIN_PER_TILE_A, WIN_PER_TILE_A)], idxbuf)
    plsc.subcore_barrier()

    @pl.loop(0, WIN_PER_TILE_A)
    def _scat(j):
        pltpu.sync_copy(ones, hist.at[idxbuf.at[j]], add=True)

    plsc.subcore_barrier()

    @pl.when(s < NS - 1)
    def _wb():
        pltpu.sync_copy(hist.at[pl.ds(s * 640, 640)], deg.at[c, pl.ds(s * 640, 640)])

    @pl.when(s == NS - 1)
    def _wb_last():
        pltpu.sync_copy(hist.at[pl.ds(9600, 400)], deg.at[c, pl.ds(9600, 400)])


_AGG_KERNEL_ARGS = dict(
    out_type=jax.ShapeDtypeStruct((NC, N, F), jnp.float32),
    mesh=_mesh,
    scratch_types=[
        pltpu.VMEM_SHARED((N, F), jnp.float32),
        pltpu.VMEM((WIN_PER_TILE_C, WIN), jnp.int32),
        pltpu.VMEM((WIN_PER_TILE_C, WIN), jnp.int32),
        pltpu.VMEM((2, WIN, F), jnp.float32),
        pltpu.SemaphoreType.DMA((2,)),
        pltpu.SemaphoreType.DMA((2,)),
    ],
    compiler_params=pltpu.CompilerParams(use_tc_tiling_on_sc=False),
)


def _aggregate_body(h2, e3, zeros_h, acc_out, accs, sidx, didx, rbuf, gsem, ssem):
    c = lax.axis_index("c")
    s = lax.axis_index("s")
    base = c * (NWIN // NC) + s * WIN_PER_TILE_C

    pltpu.sync_copy(zeros_h.at[pl.ds(s * ROWS_PER_TILE, ROWS_PER_TILE)],
                    accs.at[pl.ds(s * ROWS_PER_TILE, ROWS_PER_TILE)])
    pltpu.sync_copy(e3.at[0, pl.ds(base, WIN_PER_TILE_C)], sidx)
    pltpu.sync_copy(e3.at[1, pl.ds(base, WIN_PER_TILE_C)], didx)
    plsc.subcore_barrier()

    @pl.loop(0, WIN_PER_TILE_C)
    def _edges(j):
        pltpu.sync_copy(h2.at[sidx.at[j]], rbuf.at[0])
        pltpu.sync_copy(rbuf.at[0], accs.at[didx.at[j]], add=True)

    plsc.subcore_barrier()
    pltpu.sync_copy(accs.at[pl.ds(s * ROWS_PER_TILE, ROWS_PER_TILE)],
                    acc_out.at[c, pl.ds(s * ROWS_PER_TILE, ROWS_PER_TILE)])


def _matmul_body(x_ref, w_ref, d_ref, o_ref):
    m = jnp.dot(x_ref[...], w_ref[...], preferred_element_type=jnp.float32)
    scale = lax.rsqrt(jnp.maximum(d_ref[...], 1.0))
    o_ref[...] = m * scale


def _scaled_matmul(x, w, deg_out):
    blk = 400
    grid = N // blk
    return pl.pallas_call(
        _matmul_body,
        grid=(grid,),
        in_specs=[
            pl.BlockSpec((blk, D), lambda i: (i, 0)),
            pl.BlockSpec((D, F), lambda i: (0, 0)),
            pl.BlockSpec((blk, 1), lambda i: (i, 0)),
        ],
        out_specs=pl.BlockSpec((blk, F), lambda i: (i, 0)),
        out_shape=jax.ShapeDtypeStruct((N, F), jnp.float32),
    )(x, w, deg_out)


def _final_body(a_ref, d_ref, o_ref):
    a = a_ref[0] + a_ref[1]
    scale = lax.rsqrt(jnp.maximum(d_ref[...], 1.0))
    o_ref[...] = jnp.maximum(a * scale, 0.0)


def _finalize(acc, deg_in):
    blk = 400
    grid = N // blk
    return pl.pallas_call(
        _final_body,
        grid=(grid,),
        in_specs=[
            pl.BlockSpec((2, blk, F), lambda i: (0, i, 0)),
            pl.BlockSpec((blk, 1), lambda i: (i, 0)),
        ],
        out_specs=pl.BlockSpec((blk, F), lambda i: (i, 0)),
        out_shape=jax.ShapeDtypeStruct((N, F), jnp.float32),
    )(acc, deg_in)


_degree_kernel = pl.kernel(_degree_body, **_DEG_KERNEL_ARGS)
_aggregate_kernel = pl.kernel(_aggregate_body, **_AGG_KERNEL_ARGS)


def kernel(x, edge_index, W):
    e3 = edge_index.reshape(2, NWIN, WIN)
    deg = _degree_kernel(e3)
    h2 = _scaled_matmul(x, W, deg[0].reshape(N, 1))
    zeros_h = jnp.zeros((N, F), jnp.float32)
    acc = _aggregate_kernel(h2, e3, zeros_h)
    return _finalize(acc, deg[1].reshape(N, 1))


# speedup vs baseline: 31.7828x; 1.3200x over previous
import functools

import jax
import jax.numpy as jnp
from jax import lax
from jax.experimental import pallas as pl
from jax.experimental.pallas import tpu as pltpu
from jax.experimental.pallas import tpu_sc as plsc

N = 10000
E = 320000
D = 128
F = 128

NC = 2
NS = 16
WIN = 80
NWIN = E // WIN
WIN_PER_TILE_A = NWIN // NS
WIN_PER_TILE_C = NWIN // (NC * NS)
NPAD = 10240
ROWS_PER_TILE = N // NS

_mesh = plsc.VectorSubcoreMesh(core_axis_name="c", subcore_axis_name="s",
                               num_cores=NC, num_subcores=NS)


_DEG_KERNEL_ARGS = dict(
    out_type=jax.ShapeDtypeStruct((2, N), jnp.float32),
    mesh=_mesh,
    scratch_types=[
        pltpu.VMEM_SHARED((NPAD,), jnp.float32),
        pltpu.VMEM((WIN_PER_TILE_A, WIN), jnp.int32),
        pltpu.VMEM((WIN,), jnp.float32),
        pltpu.VMEM((640,), jnp.float32),
        pltpu.SemaphoreType.DMA((5,)),
    ],
    compiler_params=pltpu.CompilerParams(use_tc_tiling_on_sc=False),
)


def _degree_body(e3, deg, hist, idxbuf, ones, zbuf, sems):
    c = lax.axis_index("c")
    s = lax.axis_index("s")

    @pl.loop(0, 640 // 16)
    def _fill_z(i):
        zbuf[pl.ds(i * 16, 16)] = jnp.zeros((16,), jnp.float32)

    @pl.loop(0, WIN // 16)
    def _fill_o(i):
        ones[pl.ds(i * 16, 16)] = jnp.ones((16,), jnp.float32)

    pltpu.sync_copy(zbuf, hist.at[pl.ds(s * 640, 640)])
    pltpu.sync_copy(e3.at[c, pl.ds(s * WIN_PER_TILE_A, WIN_PER_TILE_A)], idxbuf)
    plsc.subcore_barrier()

    @pl.loop(0, WIN_PER_TILE_A, step=5)
    def _scat(j):
        cps = [
            pltpu.async_copy(ones, hist.at[idxbuf.at[j + b]], sems.at[b], add=True)
            for b in range(5)
        ]
        for cp in cps:
            cp.wait()

    plsc.subcore_barrier()

    @pl.when(s < NS - 1)
    def _wb():
        pltpu.sync_copy(hist.at[pl.ds(s * 640, 640)], deg.at[c, pl.ds(s * 640, 640)])

    @pl.when(s == NS - 1)
    def _wb_last():
        pltpu.sync_copy(hist.at[pl.ds(9600, 400)], deg.at[c, pl.ds(9600, 400)])


_AGG_KERNEL_ARGS = dict(
    out_type=jax.ShapeDtypeStruct((NC, N, F), jnp.float32),
    mesh=_mesh,
    scratch_types=[
        pltpu.VMEM_SHARED((N, F), jnp.float32),
        pltpu.VMEM((WIN_PER_TILE_C, WIN), jnp.int32),
        pltpu.VMEM((WIN_PER_TILE_C, WIN), jnp.int32),
        pltpu.VMEM((3, WIN, F), jnp.float32),
        pltpu.SemaphoreType.DMA((3,)),
        pltpu.SemaphoreType.DMA((3,)),
    ],
    compiler_params=pltpu.CompilerParams(use_tc_tiling_on_sc=False),
)


def _aggregate_body(h2, e3, zeros_h, acc_out, accs, sidx, didx, rbuf, gsem, ssem):
    c = lax.axis_index("c")
    s = lax.axis_index("s")
    base = c * (NWIN // NC) + s * WIN_PER_TILE_C

    pltpu.sync_copy(zeros_h.at[pl.ds(s * ROWS_PER_TILE, ROWS_PER_TILE)],
                    accs.at[pl.ds(s * ROWS_PER_TILE, ROWS_PER_TILE)])
    pltpu.sync_copy(e3.at[0, pl.ds(base, WIN_PER_TILE_C)], sidx)
    pltpu.sync_copy(e3.at[1, pl.ds(base, WIN_PER_TILE_C)], didx)
    plsc.subcore_barrier()

    @pl.loop(0, WIN_PER_TILE_C - 2, step=3)
    def _edges(j):
        gcps = [
            pltpu.async_copy(h2.at[sidx.at[j + b]], rbuf.at[b], gsem.at[b])
            for b in range(3)
        ]
        scps = []
        for b in range(3):
            gcps[b].wait()
            scps.append(
                pltpu.async_copy(rbuf.at[b], accs.at[didx.at[j + b]],
                                 ssem.at[b], add=True)
            )
        for cp in scps:
            cp.wait()

    for t in (2, 1):
        last = WIN_PER_TILE_C - t
        pltpu.sync_copy(h2.at[sidx.at[last]], rbuf.at[0])
        pltpu.sync_copy(rbuf.at[0], accs.at[didx.at[last]], add=True)

    plsc.subcore_barrier()
    pltpu.sync_copy(accs.at[pl.ds(s * ROWS_PER_TILE, ROWS_PER_TILE)],
                    acc_out.at[c, pl.ds(s * ROWS_PER_TILE, ROWS_PER_TILE)])


def _matmul_body(x_ref, w_ref, d_ref, o_ref):
    m = jnp.dot(x_ref[...], w_ref[...], preferred_element_type=jnp.float32)
    scale = lax.rsqrt(jnp.maximum(d_ref[...], 1.0))
    o_ref[...] = m * scale


def _scaled_matmul(x, w, deg_out):
    blk = 400
    grid = N // blk
    return pl.pallas_call(
        _matmul_body,
        grid=(grid,),
        in_specs=[
            pl.BlockSpec((blk, D), lambda i: (i, 0)),
            pl.BlockSpec((D, F), lambda i: (0, 0)),
            pl.BlockSpec((blk, 1), lambda i: (i, 0)),
        ],
        out_specs=pl.BlockSpec((blk, F), lambda i: (i, 0)),
        out_shape=jax.ShapeDtypeStruct((N, F), jnp.float32),
    )(x, w, deg_out)


def _final_body(a_ref, d_ref, o_ref):
    a = a_ref[0] + a_ref[1]
    scale = lax.rsqrt(jnp.maximum(d_ref[...], 1.0))
    o_ref[...] = jnp.maximum(a * scale, 0.0)


def _finalize(acc, deg_in):
    blk = 400
    grid = N // blk
    return pl.pallas_call(
        _final_body,
        grid=(grid,),
        in_specs=[
            pl.BlockSpec((2, blk, F), lambda i: (0, i, 0)),
            pl.BlockSpec((blk, 1), lambda i: (i, 0)),
        ],
        out_specs=pl.BlockSpec((blk, F), lambda i: (i, 0)),
        out_shape=jax.ShapeDtypeStruct((N, F), jnp.float32),
    )(acc, deg_in)


_degree_kernel = pl.kernel(_degree_body, **_DEG_KERNEL_ARGS)
_aggregate_kernel = pl.kernel(_aggregate_body, **_AGG_KERNEL_ARGS)


def kernel(x, edge_index, W):
    e3 = edge_index.reshape(2, NWIN, WIN)
    deg = _degree_kernel(e3)
    h2 = _scaled_matmul(x, W, deg[0].reshape(N, 1))
    zeros_h = jnp.zeros((N, F), jnp.float32)
    acc = _aggregate_kernel(h2, e3, zeros_h)
    return _finalize(acc, deg[1].reshape(N, 1))


# speedup vs baseline: 35.7037x; 1.1234x over previous
import functools

import jax
import jax.numpy as jnp
from jax import lax
from jax.experimental import pallas as pl
from jax.experimental.pallas import tpu as pltpu
from jax.experimental.pallas import tpu_sc as plsc

N = 10000
E = 320000
D = 128
F = 128

NC = 2
NS = 16
WIN = 80
NWIN = E // WIN
WIN_PER_TILE_A = NWIN // NS
WIN_PER_TILE_C = NWIN // (NC * NS)
NPAD = 10240
ROWS_PER_TILE = N // NS

_mesh = plsc.VectorSubcoreMesh(core_axis_name="c", subcore_axis_name="s",
                               num_cores=NC, num_subcores=NS)


_DEG_KERNEL_ARGS = dict(
    out_type=jax.ShapeDtypeStruct((2, N), jnp.float32),
    mesh=_mesh,
    scratch_types=[
        pltpu.VMEM_SHARED((NPAD,), jnp.float32),
        pltpu.VMEM((WIN_PER_TILE_A, WIN), jnp.int32),
        pltpu.VMEM((WIN,), jnp.float32),
        pltpu.VMEM((640,), jnp.float32),
        pltpu.SemaphoreType.DMA((5,)),
    ],
    compiler_params=pltpu.CompilerParams(use_tc_tiling_on_sc=False),
)


def _degree_body(e3, deg, hist, idxbuf, ones, zbuf, sems):
    c = lax.axis_index("c")
    s = lax.axis_index("s")

    @pl.loop(0, 640 // 16)
    def _fill_z(i):
        zbuf[pl.ds(i * 16, 16)] = jnp.zeros((16,), jnp.float32)

    @pl.loop(0, WIN // 16)
    def _fill_o(i):
        ones[pl.ds(i * 16, 16)] = jnp.ones((16,), jnp.float32)

    pltpu.sync_copy(zbuf, hist.at[pl.ds(s * 640, 640)])
    pltpu.sync_copy(e3.at[c, pl.ds(s * WIN_PER_TILE_A, WIN_PER_TILE_A)], idxbuf)
    plsc.subcore_barrier()

    @pl.loop(0, WIN_PER_TILE_A, step=5)
    def _scat(j):
        cps = [
            pltpu.async_copy(ones, hist.at[idxbuf.at[j + b]], sems.at[b], add=True)
            for b in range(5)
        ]
        for cp in cps:
            cp.wait()

    plsc.subcore_barrier()

    @pl.when(s < NS - 1)
    def _wb():
        pltpu.sync_copy(hist.at[pl.ds(s * 640, 640)], deg.at[c, pl.ds(s * 640, 640)])

    @pl.when(s == NS - 1)
    def _wb_last():
        pltpu.sync_copy(hist.at[pl.ds(9600, 400)], deg.at[c, pl.ds(9600, 400)])


_AGG_KERNEL_ARGS = dict(
    out_type=jax.ShapeDtypeStruct((NC, N, F), jnp.float32),
    mesh=_mesh,
    scratch_types=[
        pltpu.VMEM_SHARED((N, F), jnp.float32),
        pltpu.VMEM((WIN_PER_TILE_C, WIN), jnp.int32),
        pltpu.VMEM((WIN_PER_TILE_C, WIN), jnp.int32),
        pltpu.VMEM((3, WIN, F), jnp.float32),
        pltpu.SemaphoreType.DMA((3,)),
        pltpu.SemaphoreType.DMA((3,)),
    ],
    compiler_params=pltpu.CompilerParams(use_tc_tiling_on_sc=False),
)


def _aggregate_body(h2, e3, zeros_h, acc_out, accs, sidx, didx, rbuf, gsem, ssem):
    c = lax.axis_index("c")
    s = lax.axis_index("s")
    base = c * (NWIN // NC) + s * WIN_PER_TILE_C

    pltpu.sync_copy(zeros_h.at[pl.ds(s * ROWS_PER_TILE, ROWS_PER_TILE)],
                    accs.at[pl.ds(s * ROWS_PER_TILE, ROWS_PER_TILE)])
    pltpu.sync_copy(e3.at[0, pl.ds(base, WIN_PER_TILE_C)], sidx)
    pltpu.sync_copy(e3.at[1, pl.ds(base, WIN_PER_TILE_C)], didx)
    plsc.subcore_barrier()

    @pl.loop(0, WIN_PER_TILE_C - 2, step=3)
    def _edges(j):
        for b in range(3):
            @pl.when(j >= 3)
            def _drain_prev():
                pltpu.make_async_copy(
                    rbuf.at[b], accs.at[didx.at[j - 3 + b]], ssem.at[b]
                ).wait()

            pltpu.async_copy(h2.at[sidx.at[j + b]], rbuf.at[b], gsem.at[b])

        for b in range(3):
            pltpu.make_async_copy(
                h2.at[sidx.at[j + b]], rbuf.at[b], gsem.at[b]
            ).wait()
            pltpu.async_copy(rbuf.at[b], accs.at[didx.at[j + b]],
                             ssem.at[b], add=True)

    for b in range(3):
        pltpu.make_async_copy(
            rbuf.at[b], accs.at[didx.at[120 + b]], ssem.at[b]
        ).wait()

    for t in (2, 1):
        last = WIN_PER_TILE_C - t
        pltpu.sync_copy(h2.at[sidx.at[last]], rbuf.at[0])
        pltpu.sync_copy(rbuf.at[0], accs.at[didx.at[last]], add=True)

    plsc.subcore_barrier()
    pltpu.sync_copy(accs.at[pl.ds(s * ROWS_PER_TILE, ROWS_PER_TILE)],
                    acc_out.at[c, pl.ds(s * ROWS_PER_TILE, ROWS_PER_TILE)])


def _matmul_body(x_ref, w_ref, d_ref, o_ref):
    m = jnp.dot(x_ref[...], w_ref[...], preferred_element_type=jnp.float32)
    scale = lax.rsqrt(jnp.maximum(d_ref[...], 1.0))
    o_ref[...] = m * scale


def _scaled_matmul(x, w, deg_out):
    blk = 400
    grid = N // blk
    return pl.pallas_call(
        _matmul_body,
        grid=(grid,),
        in_specs=[
            pl.BlockSpec((blk, D), lambda i: (i, 0)),
            pl.BlockSpec((D, F), lambda i: (0, 0)),
            pl.BlockSpec((blk, 1), lambda i: (i, 0)),
        ],
        out_specs=pl.BlockSpec((blk, F), lambda i: (i, 0)),
        out_shape=jax.ShapeDtypeStruct((N, F), jnp.float32),
    )(x, w, deg_out)


def _final_body(a_ref, d_ref, o_ref):
    a = a_ref[0] + a_ref[1]
    scale = lax.rsqrt(jnp.maximum(d_ref[...], 1.0))
    o_ref[...] = jnp.maximum(a * scale, 0.0)


def _finalize(acc, deg_in):
    blk = 400
    grid = N // blk
    return pl.pallas_call(
        _final_body,
        grid=(grid,),
        in_specs=[
            pl.BlockSpec((2, blk, F), lambda i: (0, i, 0)),
            pl.BlockSpec((blk, 1), lambda i: (i, 0)),
        ],
        out_specs=pl.BlockSpec((blk, F), lambda i: (i, 0)),
        out_shape=jax.ShapeDtypeStruct((N, F), jnp.float32),
    )(acc, deg_in)


_degree_kernel = pl.kernel(_degree_body, **_DEG_KERNEL_ARGS)
_aggregate_kernel = pl.kernel(_aggregate_body, **_AGG_KERNEL_ARGS)


def kernel(x, edge_index, W):
    e3 = edge_index.reshape(2, NWIN, WIN)
    deg = _degree_kernel(e3)
    h2 = _scaled_matmul(x, W, deg[0].reshape(N, 1))
    zeros_h = jnp.zeros((N, F), jnp.float32)
    acc = _aggregate_kernel(h2, e3, zeros_h)
    return _finalize(acc, deg[1].reshape(N, 1))


# speedup vs baseline: 36.2769x; 1.0161x over previous
import functools

import jax
import jax.numpy as jnp
from jax import lax
from jax.experimental import pallas as pl
from jax.experimental.pallas import tpu as pltpu
from jax.experimental.pallas import tpu_sc as plsc

N = 10000
E = 320000
D = 128
F = 128

NC = 2
NS = 16
WIN = 80
NWIN = E // WIN
WIN_PER_TILE_A = NWIN // NS
WIN_PER_TILE_C = NWIN // (NC * NS)
NPAD = 10240
ROWS_PER_TILE = N // NS

_mesh = plsc.VectorSubcoreMesh(core_axis_name="c", subcore_axis_name="s",
                               num_cores=NC, num_subcores=NS)


_DEG_KERNEL_ARGS = dict(
    out_type=jax.ShapeDtypeStruct((2, N), jnp.float32),
    mesh=_mesh,
    scratch_types=[
        pltpu.VMEM_SHARED((NPAD,), jnp.float32),
        pltpu.VMEM((WIN_PER_TILE_A, WIN), jnp.int32),
        pltpu.VMEM((WIN,), jnp.float32),
        pltpu.VMEM((640,), jnp.float32),
        pltpu.SemaphoreType.DMA((10,)),
    ],
    compiler_params=pltpu.CompilerParams(use_tc_tiling_on_sc=False),
)


def _degree_body(e3, deg, hist, idxbuf, ones, zbuf, sems):
    c = lax.axis_index("c")
    s = lax.axis_index("s")

    @pl.loop(0, 640 // 16)
    def _fill_z(i):
        zbuf[pl.ds(i * 16, 16)] = jnp.zeros((16,), jnp.float32)

    @pl.loop(0, WIN // 16)
    def _fill_o(i):
        ones[pl.ds(i * 16, 16)] = jnp.ones((16,), jnp.float32)

    pltpu.sync_copy(zbuf, hist.at[pl.ds(s * 640, 640)])
    pltpu.sync_copy(e3.at[c, pl.ds(s * WIN_PER_TILE_A, WIN_PER_TILE_A)], idxbuf)
    plsc.subcore_barrier()

    @pl.loop(0, WIN_PER_TILE_A, step=10)
    def _scat(j):
        cps = [
            pltpu.async_copy(ones, hist.at[idxbuf.at[j + b]], sems.at[b], add=True)
            for b in range(10)
        ]
        for cp in cps:
            cp.wait()

    plsc.subcore_barrier()

    @pl.when(s < NS - 1)
    def _wb():
        pltpu.sync_copy(hist.at[pl.ds(s * 640, 640)], deg.at[c, pl.ds(s * 640, 640)])

    @pl.when(s == NS - 1)
    def _wb_last():
        pltpu.sync_copy(hist.at[pl.ds(9600, 400)], deg.at[c, pl.ds(9600, 400)])


_AGG_KERNEL_ARGS = dict(
    out_type=jax.ShapeDtypeStruct((NC, N, F), jnp.float32),
    mesh=_mesh,
    scratch_types=[
        pltpu.VMEM_SHARED((N, F), jnp.float32),
        pltpu.VMEM((WIN_PER_TILE_C, WIN), jnp.int32),
        pltpu.VMEM((WIN_PER_TILE_C, WIN), jnp.int32),
        pltpu.VMEM((3, WIN, F), jnp.float32),
        pltpu.SemaphoreType.DMA((3,)),
        pltpu.SemaphoreType.DMA((3,)),
    ],
    compiler_params=pltpu.CompilerParams(use_tc_tiling_on_sc=False),
)


def _aggregate_body(h2, e3, zeros_h, acc_out, accs, sidx, didx, rbuf, gsem, ssem):
    c = lax.axis_index("c")
    s = lax.axis_index("s")
    base = c * (NWIN // NC) + s * WIN_PER_TILE_C

    zcp = pltpu.async_copy(zeros_h.at[pl.ds(s * ROWS_PER_TILE, ROWS_PER_TILE)],
                           accs.at[pl.ds(s * ROWS_PER_TILE, ROWS_PER_TILE)],
                           gsem.at[0])
    pltpu.sync_copy(e3.at[0, pl.ds(base, WIN_PER_TILE_C)], sidx)
    pltpu.sync_copy(e3.at[1, pl.ds(base, WIN_PER_TILE_C)], didx)
    zcp.wait()
    plsc.subcore_barrier()

    @pl.loop(0, WIN_PER_TILE_C - 2, step=3)
    def _edges(j):
        for b in range(3):
            @pl.when(j >= 3)
            def _drain_prev():
                pltpu.make_async_copy(
                    rbuf.at[b], accs.at[didx.at[j - 3 + b]], ssem.at[b]
                ).wait()

            pltpu.async_copy(h2.at[sidx.at[j + b]], rbuf.at[b], gsem.at[b])

        for b in range(3):
            pltpu.make_async_copy(
                h2.at[sidx.at[j + b]], rbuf.at[b], gsem.at[b]
            ).wait()
            pltpu.async_copy(rbuf.at[b], accs.at[didx.at[j + b]],
                             ssem.at[b], add=True)

    for b in range(3):
        pltpu.make_async_copy(
            rbuf.at[b], accs.at[didx.at[120 + b]], ssem.at[b]
        ).wait()

    for t in (2, 1):
        last = WIN_PER_TILE_C - t
        pltpu.sync_copy(h2.at[sidx.at[last]], rbuf.at[0])
        pltpu.sync_copy(rbuf.at[0], accs.at[didx.at[last]], add=True)

    plsc.subcore_barrier()
    pltpu.sync_copy(accs.at[pl.ds(s * ROWS_PER_TILE, ROWS_PER_TILE)],
                    acc_out.at[c, pl.ds(s * ROWS_PER_TILE, ROWS_PER_TILE)])


def _matmul_body(x_ref, w_ref, d_ref, o_ref):
    m = jnp.dot(x_ref[...], w_ref[...], preferred_element_type=jnp.float32)
    scale = lax.rsqrt(jnp.maximum(d_ref[...], 1.0))
    o_ref[...] = m * scale


def _scaled_matmul(x, w, deg_out):
    blk = 400
    grid = N // blk
    return pl.pallas_call(
        _matmul_body,
        grid=(grid,),
        in_specs=[
            pl.BlockSpec((blk, D), lambda i: (i, 0)),
            pl.BlockSpec((D, F), lambda i: (0, 0)),
            pl.BlockSpec((blk, 1), lambda i: (i, 0)),
        ],
        out_specs=pl.BlockSpec((blk, F), lambda i: (i, 0)),
        out_shape=jax.ShapeDtypeStruct((N, F), jnp.float32),
    )(x, w, deg_out)


def _final_body(a_ref, d_ref, o_ref):
    a = a_ref[0] + a_ref[1]
    scale = lax.rsqrt(jnp.maximum(d_ref[...], 1.0))
    o_ref[...] = jnp.maximum(a * scale, 0.0)


def _finalize(acc, deg_in):
    blk = 400
    grid = N // blk
    return pl.pallas_call(
        _final_body,
        grid=(grid,),
        in_specs=[
            pl.BlockSpec((2, blk, F), lambda i: (0, i, 0)),
            pl.BlockSpec((blk, 1), lambda i: (i, 0)),
        ],
        out_specs=pl.BlockSpec((blk, F), lambda i: (i, 0)),
        out_shape=jax.ShapeDtypeStruct((N, F), jnp.float32),
    )(acc, deg_in)


_degree_kernel = pl.kernel(_degree_body, **_DEG_KERNEL_ARGS)
_aggregate_kernel = pl.kernel(_aggregate_body, **_AGG_KERNEL_ARGS)


def kernel(x, edge_index, W):
    e3 = edge_index.reshape(2, NWIN, WIN)
    deg = _degree_kernel(e3)
    h2 = _scaled_matmul(x, W, deg[0].reshape(N, 1))
    zeros_h = jnp.zeros((N, F), jnp.float32)
    acc = _aggregate_kernel(h2, e3, zeros_h)
    return _finalize(acc, deg[1].reshape(N, 1))


# speedup vs baseline: 40.0903x; 1.1051x over previous
import functools

import jax
import jax.numpy as jnp
from jax import lax
from jax.experimental import pallas as pl
from jax.experimental.pallas import tpu as pltpu
from jax.experimental.pallas import tpu_sc as plsc

N = 10000
E = 320000
D = 128
F = 128

NC = 2
NS = 16
WIN = 80
NWIN = E // WIN
WIN_PER_TILE_A = NWIN // NS
WIN_PER_TILE_C = NWIN // (NC * NS)
NPAD = 10240
ROWS_PER_TILE = N // NS

_mesh = plsc.VectorSubcoreMesh(core_axis_name="c", subcore_axis_name="s",
                               num_cores=NC, num_subcores=NS)


_DEG_KERNEL_ARGS = dict(
    out_type=jax.ShapeDtypeStruct((2, N), jnp.float32),
    mesh=_mesh,
    scratch_types=[
        pltpu.VMEM_SHARED((NPAD,), jnp.float32),
        pltpu.VMEM((WIN_PER_TILE_A, WIN), jnp.int32),
        pltpu.VMEM((WIN,), jnp.float32),
        pltpu.VMEM((640,), jnp.float32),
        pltpu.SemaphoreType.DMA((10,)),
    ],
    compiler_params=pltpu.CompilerParams(use_tc_tiling_on_sc=False),
)


def _degree_body(e3, deg, hist, idxbuf, ones, zbuf, sems):
    c = lax.axis_index("c")
    s = lax.axis_index("s")

    @pl.loop(0, 640 // 16)
    def _fill_z(i):
        zbuf[pl.ds(i * 16, 16)] = jnp.zeros((16,), jnp.float32)

    @pl.loop(0, WIN // 16)
    def _fill_o(i):
        ones[pl.ds(i * 16, 16)] = jnp.ones((16,), jnp.float32)

    pltpu.sync_copy(zbuf, hist.at[pl.ds(s * 640, 640)])
    pltpu.sync_copy(e3.at[c, pl.ds(s * WIN_PER_TILE_A, WIN_PER_TILE_A)], idxbuf)
    plsc.subcore_barrier()

    @pl.loop(0, WIN_PER_TILE_A, step=10)
    def _scat(j):
        cps = [
            pltpu.async_copy(ones, hist.at[idxbuf.at[j + b]], sems.at[b], add=True)
            for b in range(10)
        ]
        for cp in cps:
            cp.wait()

    plsc.subcore_barrier()

    @pl.when(s < NS - 1)
    def _wb():
        pltpu.sync_copy(hist.at[pl.ds(s * 640, 640)], deg.at[c, pl.ds(s * 640, 640)])

    @pl.when(s == NS - 1)
    def _wb_last():
        pltpu.sync_copy(hist.at[pl.ds(9600, 400)], deg.at[c, pl.ds(9600, 400)])


_AGG_KERNEL_ARGS = dict(
    out_type=jax.ShapeDtypeStruct((NC, N, F), jnp.float32),
    mesh=_mesh,
    scratch_types=[
        pltpu.VMEM_SHARED((N, F), jnp.float32),
        pltpu.VMEM((WIN_PER_TILE_C, WIN), jnp.int32),
        pltpu.VMEM((WIN_PER_TILE_C, WIN), jnp.int32),
        pltpu.VMEM((3, WIN, F), jnp.float32),
        pltpu.SemaphoreType.DMA((3,)),
        pltpu.SemaphoreType.DMA((3,)),
    ],
    compiler_params=pltpu.CompilerParams(use_tc_tiling_on_sc=False),
)


def _aggregate_body(h2, e3, zeros_h, acc_out, accs, sidx, didx, rbuf, gsem, ssem):
    c = lax.axis_index("c")
    s = lax.axis_index("s")
    base = c * (NWIN // NC) + s * WIN_PER_TILE_C

    zcp = pltpu.async_copy(zeros_h.at[pl.ds(s * ROWS_PER_TILE, ROWS_PER_TILE)],
                           accs.at[pl.ds(s * ROWS_PER_TILE, ROWS_PER_TILE)],
                           gsem.at[0])
    pltpu.sync_copy(e3.at[0, pl.ds(base, WIN_PER_TILE_C)], sidx)
    pltpu.sync_copy(e3.at[1, pl.ds(base, WIN_PER_TILE_C)], didx)
    zcp.wait()
    plsc.subcore_barrier()

    @pl.loop(0, WIN_PER_TILE_C - 2, step=3)
    def _edges(j):
        for b in range(3):
            @pl.when(j >= 3)
            def _drain_prev():
                pltpu.make_async_copy(
                    rbuf.at[b], accs.at[didx.at[j - 3 + b]], ssem.at[b]
                ).wait()

            pltpu.async_copy(h2.at[sidx.at[j + b]], rbuf.at[b], gsem.at[b])

        for b in range(3):
            pltpu.make_async_copy(
                h2.at[sidx.at[j + b]], rbuf.at[b], gsem.at[b]
            ).wait()
            pltpu.async_copy(rbuf.at[b], accs.at[didx.at[j + b]],
                             ssem.at[b], add=True)

    for b in range(3):
        pltpu.make_async_copy(
            rbuf.at[b], accs.at[didx.at[120 + b]], ssem.at[b]
        ).wait()

    for t in (2, 1):
        last = WIN_PER_TILE_C - t
        pltpu.sync_copy(h2.at[sidx.at[last]], rbuf.at[0])
        pltpu.sync_copy(rbuf.at[0], accs.at[didx.at[last]], add=True)

    plsc.subcore_barrier()
    pltpu.sync_copy(accs.at[pl.ds(s * ROWS_PER_TILE, ROWS_PER_TILE)],
                    acc_out.at[c, pl.ds(s * ROWS_PER_TILE, ROWS_PER_TILE)])


def _matmul_body(x_ref, w_ref, d_ref, o_ref):
    m = jnp.dot(x_ref[...], w_ref[...], preferred_element_type=jnp.float32)
    scale = lax.rsqrt(jnp.maximum(d_ref[...], 1.0))
    o_ref[...] = m * scale


def _scaled_matmul(x, w, deg_out):
    blk = 2000
    grid = N // blk
    return pl.pallas_call(
        _matmul_body,
        grid=(grid,),
        in_specs=[
            pl.BlockSpec((blk, D), lambda i: (i, 0)),
            pl.BlockSpec((D, F), lambda i: (0, 0)),
            pl.BlockSpec((blk, 1), lambda i: (i, 0)),
        ],
        out_specs=pl.BlockSpec((blk, F), lambda i: (i, 0)),
        out_shape=jax.ShapeDtypeStruct((N, F), jnp.float32),
    )(x, w, deg_out)


def _final_body(a_ref, d_ref, o_ref):
    a = a_ref[0] + a_ref[1]
    scale = lax.rsqrt(jnp.maximum(d_ref[...], 1.0))
    o_ref[...] = jnp.maximum(a * scale, 0.0)


def _finalize(acc, deg_in):
    blk = 2000
    grid = N // blk
    return pl.pallas_call(
        _final_body,
        grid=(grid,),
        in_specs=[
            pl.BlockSpec((2, blk, F), lambda i: (0, i, 0)),
            pl.BlockSpec((blk, 1), lambda i: (i, 0)),
        ],
        out_specs=pl.BlockSpec((blk, F), lambda i: (i, 0)),
        out_shape=jax.ShapeDtypeStruct((N, F), jnp.float32),
    )(acc, deg_in)


_degree_kernel = pl.kernel(_degree_body, **_DEG_KERNEL_ARGS)
_aggregate_kernel = pl.kernel(_aggregate_body, **_AGG_KERNEL_ARGS)


def kernel(x, edge_index, W):
    e3 = edge_index.reshape(2, NWIN, WIN)
    deg = _degree_kernel(e3)
    h2 = _scaled_matmul(x, W, deg[0].reshape(N, 1))
    zeros_h = jnp.zeros((N, F), jnp.float32)
    acc = _aggregate_kernel(h2, e3, zeros_h)
    return _finalize(acc, deg[1].reshape(N, 1))
